# baseline (speedup 1.0000x reference)
import functools

import jax
import jax.numpy as jnp
from jax import lax
from jax.experimental import pallas as pl
from jax.experimental.pallas import tpu as pltpu
from jax.experimental.pallas import tpu_sc as plsc

N_ROWS = 16384
N_COLS = 26
EMB = 64
TOTAL = N_ROWS * N_COLS
NUM_CORES = 2
NUM_SUBCORES = 16
NW = NUM_CORES * NUM_SUBCORES
PER_W = TOTAL // NW
CHUNK = 128
NCHUNK = PER_W // CHUNK


@jax.jit
def _gather_sc(idx_flat, table):
    mesh = plsc.VectorSubcoreMesh(
        core_axis_name="c", subcore_axis_name="s",
        num_cores=NUM_CORES, num_subcores=NUM_SUBCORES)

    @functools.partial(
        pl.kernel,
        mesh=mesh,
        out_type=jax.ShapeDtypeStruct((TOTAL, EMB), jnp.float32),
        scratch_types=[
            pltpu.VMEM((NCHUNK, CHUNK), jnp.int32),
            pltpu.VMEM((CHUNK, EMB), jnp.float32),
            pltpu.SemaphoreType.DMA,
        ],
        compiler_params=pltpu.CompilerParams(use_tc_tiling_on_sc=False),
    )
    def k(idx_hbm, table_hbm, out_hbm, idx_v, rows_v, gsem):
        wid = lax.axis_index("s") * NUM_CORES + lax.axis_index("c")
        base = wid * PER_W
        pltpu.sync_copy(idx_hbm.at[wid], idx_v)

        def body(j, carry):
            pltpu.async_copy(table_hbm.at[idx_v.at[j]], rows_v, gsem).wait()
            pltpu.sync_copy(rows_v, out_hbm.at[pl.ds(base + j * CHUNK, CHUNK)])
            return carry

        lax.fori_loop(0, NCHUNK, body, 0)

    return k(idx_flat, table)


def kernel(idx, table):
    idx_flat = idx.reshape(NW, NCHUNK, CHUNK).astype(jnp.int32)
    out = _gather_sc(idx_flat, table)
    return out.reshape(N_ROWS, N_COLS, EMB)


# speedup vs baseline: 1.0801x; 1.0801x over previous
import functools

import jax
import jax.numpy as jnp
from jax import lax
from jax.experimental import pallas as pl
from jax.experimental.pallas import tpu as pltpu
from jax.experimental.pallas import tpu_sc as plsc

N_ROWS = 16384
N_COLS = 26
EMB = 64
TOTAL = N_ROWS * N_COLS
NUM_CORES = 2
NUM_SUBCORES = 16
NW = NUM_CORES * NUM_SUBCORES
PER_W = TOTAL // NW
CHUNK = 128
NCHUNK = PER_W // CHUNK
NBUF = 8
LAG = 4


@jax.jit
def _gather_sc(idx_flat, table):
    mesh = plsc.VectorSubcoreMesh(
        core_axis_name="c", subcore_axis_name="s",
        num_cores=NUM_CORES, num_subcores=NUM_SUBCORES)

    @functools.partial(
        pl.kernel,
        mesh=mesh,
        out_type=jax.ShapeDtypeStruct((TOTAL, EMB), jnp.float32),
        scratch_types=[
            pltpu.VMEM((NCHUNK, CHUNK), jnp.int32),
            pltpu.VMEM((NBUF, CHUNK, EMB), jnp.float32),
            pltpu.SemaphoreType.DMA((NBUF,)),
            pltpu.SemaphoreType.DMA((NBUF,)),
        ],
        compiler_params=pltpu.CompilerParams(use_tc_tiling_on_sc=False),
    )
    def k(idx_hbm, table_hbm, out_hbm, idx_v, rows_v, gsem, wsem):
        wid = lax.axis_index("s") * NUM_CORES + lax.axis_index("c")
        base = wid * PER_W
        pltpu.sync_copy(idx_hbm.at[wid], idx_v)

        NTOT = NCHUNK + NBUF

        @pl.loop(0, NTOT, step=NBUF)
        def _steps(j0):
            for b in range(NBUF):
                j = j0 + b

                @pl.when(j < NCHUNK)
                def _gather_stage():
                    @pl.when(j >= NBUF)
                    def _reuse_wait():
                        pltpu.make_async_copy(
                            rows_v.at[b],
                            out_hbm.at[pl.ds(base, CHUNK)],
                            wsem.at[b]).wait()
                    pltpu.async_copy(
                        table_hbm.at[idx_v.at[j]], rows_v.at[b], gsem.at[b])

                jw = j - LAG
                bw = (b - LAG) % NBUF

                @pl.when(jnp.logical_and(jw >= 0, jw < NCHUNK))
                def _write_stage():
                    pltpu.make_async_copy(
                        table_hbm.at[idx_v.at[0]],
                        rows_v.at[bw], gsem.at[bw]).wait()
                    pltpu.async_copy(
                        rows_v.at[bw],
                        out_hbm.at[pl.ds(base + jw * CHUNK, CHUNK)],
                        wsem.at[bw])

        for b in range(NBUF):
            pltpu.make_async_copy(
                rows_v.at[b], out_hbm.at[pl.ds(base, CHUNK)], wsem.at[b]).wait()

    return k(idx_flat, table)


def kernel(idx, table):
    idx_flat = idx.reshape(NW, NCHUNK, CHUNK).astype(jnp.int32)
    out = _gather_sc(idx_flat, table)
    return out.reshape(N_ROWS, N_COLS, EMB)
